# Initial kernel scaffold; baseline (speedup 1.0000x reference)
import jax
import jax.numpy as jnp
from jax.experimental import pallas as pl


def kernel(inputs, w):
    raise NotImplementedError("write your pallas kernel here")


# speedup vs baseline: 1.3101x; 1.3101x over previous
import functools

import jax
import jax.numpy as jnp
from jax import lax
from jax.experimental import pallas as pl
from jax.experimental.pallas import tpu as pltpu
from jax.experimental.pallas import tpu_sc as plsc

_BATCH = 16384
_N_FIELDS = 26
_NW = 32
_BPW = _BATCH // _NW
_IPW = _BPW * _N_FIELDS
_CHUNK = 128
_NCHUNK = _IPW // _CHUNK
_FIRE = 8
_L = 16


def _sc_body(idx_hbm, w_hbm, out_hbm, idx_v, g_v, out_v, sem):
    wid = lax.axis_index("s") * 2 + lax.axis_index("c")

    pltpu.sync_copy(idx_hbm.at[wid], idx_v)

    def gather_step(j, _):
        copies = []
        for b in range(_FIRE):
            chunk = j * _FIRE + b
            off = pl.multiple_of(chunk * _CHUNK, _CHUNK)
            copies.append(
                pltpu.async_copy(
                    w_hbm.at[idx_v.at[chunk]],
                    g_v.at[pl.ds(off, _CHUNK)],
                    sem,
                )
            )
        for c in copies:
            c.wait()
        return 0

    lax.fori_loop(0, _NCHUNK // _FIRE, gather_step, 0)

    def reduce_step(g, _):
        base = pl.multiple_of(g * _L, _L)
        acc = jnp.zeros((_L,), jnp.float32)
        for f in range(_N_FIELDS):
            acc = acc + g_v[pl.ds(f * _BPW + base, _L)]
        out_v[pl.ds(base, _L)] = acc
        return 0

    lax.fori_loop(0, _BPW // _L, reduce_step, 0)

    pltpu.sync_copy(out_v, out_hbm.at[wid])


@jax.jit
def _run(idx_blocks, w_flat):
    mesh = plsc.VectorSubcoreMesh(core_axis_name="c", subcore_axis_name="s")
    return pl.kernel(
        _sc_body,
        out_type=jax.ShapeDtypeStruct((_NW, _BPW), jnp.float32),
        mesh=mesh,
        scratch_types=[
            pltpu.VMEM((_NCHUNK, _CHUNK), jnp.int32),
            pltpu.VMEM((_IPW,), jnp.float32),
            pltpu.VMEM((_BPW,), jnp.float32),
            pltpu.SemaphoreType.DMA,
        ],
    )(idx_blocks, w_flat)


def kernel(inputs, w):
    idx_blocks = (
        inputs.reshape(_NW, _BPW, _N_FIELDS)
        .transpose(0, 2, 1)
        .reshape(_NW, _NCHUNK, _CHUNK)
    )
    out = _run(idx_blocks, w.reshape(-1))
    return out.reshape(_BATCH, 1)
